# baseline (speedup 1.0000x reference)
import functools

import jax
import jax.numpy as jnp
from jax import lax
from jax.experimental import pallas as pl
from jax.experimental.pallas import tpu as pltpu
from jax.experimental.pallas import tpu_sc as plsc

N = 10000
D = 128
E = 320000
EA = E + N
L = 5
NW = 32
CHUNK = 128
NCH = EA // CHUNK
TAIL = EA - NCH * CHUNK
NB = 5
BLK = N // NB
ECR = 64

_mesh = plsc.VectorSubcoreMesh(core_axis_name="c", subcore_axis_name="s")


def _init_h_body(x_ref, e1_ref, e2_ref, out_ref):
    x = x_ref[...]
    i0 = lax.broadcasted_iota(jnp.int32, (BLK, D), 1)
    oh1 = jnp.where(x[:, 0:1] == i0, 1.0, 0.0)
    oh2 = jnp.where(x[:, 1:2] == i0, 1.0, 0.0)
    a = jax.lax.dot(oh1, e1_ref[...], precision=lax.Precision.HIGHEST,
                    preferred_element_type=jnp.float32)
    b = jax.lax.dot(oh2, e2_ref[...], precision=lax.Precision.HIGHEST,
                    preferred_element_type=jnp.float32)
    out_ref[...] = a + b


def _init_h(x, e1p, e2p):
    return pl.pallas_call(
        _init_h_body,
        grid=(NB,),
        in_specs=[
            pl.BlockSpec((BLK, 2), lambda i: (i, 0)),
            pl.BlockSpec((D, D), lambda i: (0, 0)),
            pl.BlockSpec((D, D), lambda i: (0, 0)),
        ],
        out_specs=pl.BlockSpec((BLK, D), lambda i: (i, 0)),
        out_shape=jax.ShapeDtypeStruct((N, D), jnp.float32),
    )(x, e1p, e2p)


@functools.partial(
    pl.kernel,
    mesh=_mesh,
    out_type=jax.ShapeDtypeStruct((EA, D), jnp.float32),
    scratch_types=[
        pltpu.VMEM((CHUNK,), jnp.int32),
        pltpu.VMEM((CHUNK,), jnp.int32),
        pltpu.VMEM((CHUNK,), jnp.int32),
        pltpu.VMEM((CHUNK,), jnp.int32),
        pltpu.VMEM((2, CHUNK, D), jnp.float32),
        pltpu.VMEM((2, CHUNK, D), jnp.float32),
        pltpu.SemaphoreType.DMA,
        pltpu.SemaphoreType.DMA,
        pltpu.SemaphoreType.DMA,
        pltpu.SemaphoreType.DMA,
    ],
)
def _msg_build(h_hbm, ecomb_hbm, src_hbm, c_hbm, out_hbm,
               src0_v, src1_v, c0_v, c1_v, hrow_v, erow_v,
               sem_h0, sem_h1, sem_e0, sem_e1):
    cc = lax.axis_index("c")
    ss = lax.axis_index("s")
    w = cc * 16 + ss
    nt = (NCH + 1 - w + NW - 1) // NW

    def issue(t, src_v, c_v, sem_h, sem_e, buf):
        ci = t * NW + w
        pltpu.sync_copy(src_hbm.at[pl.ds(ci * CHUNK, CHUNK)], src_v)
        pltpu.sync_copy(c_hbm.at[pl.ds(ci * CHUNK, CHUNK)], c_v)
        pltpu.async_copy(h_hbm.at[src_v], hrow_v.at[buf], sem_h)
        pltpu.async_copy(ecomb_hbm.at[c_v], erow_v.at[buf], sem_e)

    def finish(t, src_v, c_v, sem_h, sem_e, buf):
        ci = t * NW + w
        pltpu.make_async_copy(h_hbm.at[src_v], hrow_v.at[buf], sem_h).wait()
        pltpu.make_async_copy(ecomb_hbm.at[c_v], erow_v.at[buf], sem_e).wait()

        def add_row(r, carry):
            for k in range(D // 16):
                o = k * 16
                hrow_v[buf, r, pl.ds(o, 16)] = (
                    hrow_v[buf, r, pl.ds(o, 16)] + erow_v[buf, r, pl.ds(o, 16)]
                )
            return carry

        lax.fori_loop(0, CHUNK, add_row, 0)

        @pl.when(ci < NCH)
        def _():
            pltpu.sync_copy(hrow_v.at[buf], out_hbm.at[pl.ds(ci * CHUNK, CHUNK)])

        @pl.when(ci == NCH)
        def _():
            pltpu.sync_copy(hrow_v.at[buf].at[pl.ds(0, TAIL)],
                            out_hbm.at[pl.ds(NCH * CHUNK, TAIL)])

    @pl.when(nt > 0)
    def _():
        issue(0, src0_v, c0_v, sem_h0, sem_e0, 0)

        @pl.when(nt > 1)
        def _():
            issue(1, src1_v, c1_v, sem_h1, sem_e1, 1)

        def body(t, carry):
            @pl.when(t % 2 == 0)
            def _():
                finish(t, src0_v, c0_v, sem_h0, sem_e0, 0)

                @pl.when(t + 2 < nt)
                def _():
                    issue(t + 2, src0_v, c0_v, sem_h0, sem_e0, 0)

            @pl.when(t % 2 == 1)
            def _():
                finish(t, src1_v, c1_v, sem_h1, sem_e1, 1)

                @pl.when(t + 2 < nt)
                def _():
                    issue(t + 2, src1_v, c1_v, sem_h1, sem_e1, 1)

            return carry

        lax.fori_loop(0, nt, body, 0)


def kernel(x, edge_index, edge_attr, x_emb1, x_emb2, edge_emb1, edge_emb2,
           W1, b1, W2, b2, gamma, beta):
    n = x.shape[0]
    e1p = jnp.zeros((D, D), jnp.float32).at[:x_emb1.shape[0]].set(x_emb1)
    e2p = jnp.zeros((D, D), jnp.float32).at[:x_emb2.shape[0]].set(x_emb2)
    h = _init_h(x.astype(jnp.int32), e1p, e2p)

    loop = jnp.arange(n, dtype=edge_index.dtype)
    ei = jnp.concatenate([edge_index, jnp.stack([loop, loop], axis=0)], axis=1)
    self_attr = jnp.zeros((n, 2), dtype=edge_attr.dtype).at[:, 0].set(4)
    ea = jnp.concatenate([edge_attr, self_attr], axis=0)
    src = ei[0]
    dst = ei[1]

    cidx = (ea[:, 0] * 3 + ea[:, 1]).astype(jnp.int32)
    cidx = cidx + 18 * (jnp.arange(EA, dtype=jnp.int32) % ECR)
    pad = (NCH + 1) * CHUNK - EA
    src_c = jnp.concatenate(
        [src.astype(jnp.int32), jnp.zeros((pad,), jnp.int32)])
    c_c = jnp.concatenate([cidx, jnp.zeros((pad,), jnp.int32)])

    for l in range(L):
        ecomb = (edge_emb1[l][:, None, :] + edge_emb2[l][None, :, :]
                 ).reshape(18, D)
        msg = _msg_build(h, jnp.tile(ecomb, (ECR, 1)), src_c, c_c)
        aggr = jax.ops.segment_sum(msg, dst, num_segments=n)
        hid = jnp.maximum(aggr @ W1[l] + b1[l], 0.0)
        h2 = hid @ W2[l] + b2[l]
        mean = jnp.mean(h2, axis=0)
        var = jnp.var(h2, axis=0)
        h2 = (h2 - mean) / jnp.sqrt(var + 1e-5) * gamma[l] + beta[l]
        if l < L - 1:
            h2 = jnp.maximum(h2, 0.0)
        h = h2
    return h
